# baseline (speedup 1.0000x reference)
import functools

import jax
import jax.numpy as jnp
from jax import lax
from jax.experimental import pallas as pl
from jax.experimental.pallas import tpu as pltpu
from jax.experimental.pallas import tpu_sc as plsc

_N = 1024
_M = 16384
_D = 128
_MT = 2048
_NW = 32
_RPW = _M // _NW
_CHR = 128
_NCH = _RPW // _CHR
_NOUT = _N + _NW


def _ex_body(lane_ref, wl_ref, ex_ref):
    s = jax.lax.dot_general(lane_ref[...], wl_ref[...], (((1,), (0,)), ((), ())),
                            preferred_element_type=jnp.float32)
    ex_ref[...] = jnp.exp(s)


def _tc_ex(lane, wl):
    return pl.pallas_call(
        _ex_body,
        grid=(_M // _MT,),
        in_specs=[
            pl.BlockSpec((_MT, _D), lambda i: (i, 0)),
            pl.BlockSpec((_D, 1), lambda i: (0, 0)),
        ],
        out_specs=pl.BlockSpec((_MT, 1), lambda i: (i, 0)),
        out_shape=jax.ShapeDtypeStruct((_M, 1), jnp.float32),
    )(lane, wl)


_sc_mesh = plsc.VectorSubcoreMesh(core_axis_name="c", subcore_axis_name="s")


@functools.partial(
    pl.kernel,
    out_type=(jax.ShapeDtypeStruct((_NW, _D), jnp.float32),
              jax.ShapeDtypeStruct((_NW, _D), jnp.float32),
              jax.ShapeDtypeStruct((_NW, _D), jnp.int32)),
    mesh=_sc_mesh,
    compiler_params=pltpu.CompilerParams(needs_layout_passes=False,
                                         use_tc_tiling_on_sc=False),
    scratch_types=[
        pltpu.VMEM((_CHR, _D), jnp.float32),
        pltpu.VMEM((_CHR, _D), jnp.float32),
        pltpu.VMEM((_RPW,), jnp.float32),
        pltpu.VMEM((768,), jnp.int32),
        pltpu.VMEM((_RPW,), jnp.int32),
        pltpu.VMEM((_RPW,), jnp.int32),
        pltpu.VMEM((4, _D), jnp.int32),
        pltpu.VMEM((_RPW, _D), jnp.float32),
        pltpu.VMEM((_RPW, 16), jnp.float32),
        pltpu.VMEM((1, _D), jnp.float32),
        pltpu.VMEM((1, _D), jnp.int32),
        pltpu.SemaphoreType.DMA,
        pltpu.SemaphoreType.DMA,
        pltpu.SemaphoreType.DMA,
    ],
)
def _sc_scatter(lane_hbm, idx_hbm, ex_hbm, num_ref, den_ref,
                side_num, side_den, side_id,
                lane0_b, lane1_b, ex_b, idxe_b, segs_b, ids_b, ids2_b,
                locn_b, locd_b, tmpf_b, tmpi_b, semA, semB, semC):
    c = lax.axis_index("c")
    s = lax.axis_index("s")
    w = c * 16 + s
    base = w * _RPW
    li = lax.iota(jnp.int32, 16)
    bufs = (lane0_b, lane1_b)
    sems = (semA, semB)

    handles = [None] * _NCH
    handles[0] = pltpu.async_copy(lane_hbm.at[pl.ds(base, _CHR)], lane0_b,
                                  semA)
    handles[1] = pltpu.async_copy(lane_hbm.at[pl.ds(base + _CHR, _CHR)],
                                  lane1_b, semB)
    pltpu.sync_copy(ex_hbm.at[pl.ds(base, _RPW)], ex_b)
    pltpu.sync_copy(idx_hbm.at[pl.ds(base, _RPW)], idxe_b.at[pl.ds(16, _RPW)])
    head = idxe_b[pl.ds(16, 16)]
    idxe_b[pl.ds(0, 16)] = jnp.broadcast_to(jnp.min(head), (16,))

    dumpv = jnp.full((16,), _N + w, jnp.int32)
    for k in range(_RPW // 16):
        ids_b[pl.ds(k * 16, 16)] = dumpv

    c15 = jnp.full((16,), 15, jnp.int32)

    def _seg(g, sbv):
        iv = idxe_b[pl.ds(16 + g * 16, 16)]
        sv = idxe_b[pl.ds(15 + g * 16, 16)]
        segs = sbv + plsc.cumsum((iv != sv).astype(jnp.int32))
        segs_b[pl.ds(g * 16, 16)] = segs
        plsc.store_scatter(ids_b, [segs], iv)
        return jnp.take_along_axis(segs, c15, axis=0)

    sbv = lax.fori_loop(0, _RPW // 16, _seg, jnp.zeros((16,), jnp.int32))
    nseg = jnp.max(sbv) + 1

    zv = jnp.zeros((16,), jnp.float32)

    def _zr(r, carry):
        for dd in range(_D // 16):
            locn_b[r, pl.ds(dd * 16, 16)] = zv
        locd_b[r, pl.ds(0, 16)] = zv
        return carry

    lax.fori_loop(0, nseg, _zr, 0)

    for ci in range(_NCH):
        handles[ci].wait()
        buf = bufs[ci % 2]

        def _grp(gp, carry, ci=ci, buf=buf):
            r0a = ci * _CHR + gp * 16
            r0b = r0a + _CHR // 2
            segsa = segs_b[pl.ds(r0a, 16)]
            segsb = segs_b[pl.ds(r0b, 16)]
            exva = ex_b[pl.ds(r0a, 16)]
            exvb = ex_b[pl.ds(r0b, 16)]
            for rr in range(16):
                cvec = jnp.full((16,), rr, jnp.int32)
                sega = jnp.take_along_axis(segsa, cvec, axis=0)
                evra = jnp.take_along_axis(exva, cvec, axis=0)
                segc = jnp.take_along_axis(segsb, cvec, axis=0)
                evrc = jnp.take_along_axis(exvb, cvec, axis=0)
                rla = gp * 16 + rr
                rlb = rla + _CHR // 2
                for dd in range(_D // 16):
                    va = buf[rla, pl.ds(dd * 16, 16)]
                    plsc.addupdate_scatter(locn_b, [sega, li + dd * 16],
                                           va * evra)
                    vb = buf[rlb, pl.ds(dd * 16, 16)]
                    plsc.addupdate_scatter(locn_b, [segc, li + dd * 16],
                                           vb * evrc)
                plsc.addupdate_scatter(locd_b, [sega, li], evra)
                plsc.addupdate_scatter(locd_b, [segc, li], evrc)
            return carry

        lax.fori_loop(0, _CHR // 32, _grp, 0)
        if ci + 2 < _NCH:
            handles[ci + 2] = pltpu.async_copy(
                lane_hbm.at[pl.ds(base + (ci + 2) * _CHR, _CHR)],
                bufs[ci % 2], sems[ci % 2])

    zv128 = jnp.zeros((16,), jnp.float32)
    for dd in range(_D // 16):
        tmpf_b[0, pl.ds(dd * 16, 16)] = zv128
        tmpi_b[0, pl.ds(dd * 16, 16)] = jnp.zeros((16,), jnp.int32)
    hside_n = pltpu.async_copy(locn_b.at[pl.ds(0, 1)],
                               side_num.at[pl.ds(w, 1)], semC)
    tmpf_b[0, pl.ds(0, 16)] = locd_b[0, pl.ds(0, 16)]
    hside_d = pltpu.async_copy(tmpf_b, side_den.at[pl.ds(w, 1)], semC)
    idv = ids_b[pl.ds(0, 16)]
    tmpi_b[0, pl.ds(0, 16)] = idv
    hside_i = pltpu.async_copy(tmpi_b, side_id.at[pl.ds(w, 1)], semC)
    ids_b[pl.ds(0, 16)] = jnp.where(li == 0, dumpv, idv)

    for k in range(_RPW // 16):
        ids2_b[k // 8, pl.ds((k % 8) * 16, 16)] = ids_b[pl.ds(k * 16, 16)]

    def _cp0(r, carry):
        lane0_b[r, pl.ds(0, 16)] = locd_b[r, pl.ds(0, 16)]
        return carry

    lax.fori_loop(0, 128, _cp0, 0)
    hnum = pltpu.async_copy(locn_b.at[pl.ds(0, 128)],
                            num_ref.at[ids2_b.at[0]], semC)
    hden = pltpu.async_copy(lane0_b, den_ref.at[ids2_b.at[0]], semC)

    for k in range(1, 4):
        @pl.when(nseg > k * 128)
        def _(k=k):
            pltpu.sync_copy(locn_b.at[pl.ds(k * 128, 128)],
                            num_ref.at[ids2_b.at[k]])

            def _cp(r, carry):
                lane1_b[r, pl.ds(0, 16)] = locd_b[k * 128 + r, pl.ds(0, 16)]
                return carry

            lax.fori_loop(0, 128, _cp, 0)
            pltpu.sync_copy(lane1_b, den_ref.at[ids2_b.at[k]])

    hside_n.wait()
    hside_d.wait()
    hside_i.wait()
    hnum.wait()
    hden.wait()


def _norm_body(num_ref, den_ref, sn_ref, sd_ref, si_ref, out_ref):
    num = num_ref[...][:_N]
    den = den_ref[...][:_N, 0:1]
    sid = si_ref[...][:, 0:1]
    obs = lax.broadcasted_iota(jnp.int32, (_N, _NW), 0)
    sidb = jnp.broadcast_to(sid.T, (_N, _NW))
    sel = jnp.where(obs == sidb, 1.0, 0.0)
    num_tot = num + jax.lax.dot_general(
        sel, sn_ref[...], (((1,), (0,)), ((), ())),
        preferred_element_type=jnp.float32)
    den_tot = den + jax.lax.dot_general(
        sel, sd_ref[...], (((1,), (0,)), ((), ())),
        preferred_element_type=jnp.float32)[:, 0:1]
    ok = den_tot > 0.0
    denq = jnp.where(ok, den_tot, 1.0)
    out_ref[...] = jnp.where(jnp.broadcast_to(ok, (_N, _D)),
                             num_tot / jnp.broadcast_to(denq, (_N, _D)),
                             0.0)


def _tc_norm(num, den, side_num, side_den, side_id):
    return pl.pallas_call(
        _norm_body,
        in_specs=[
            pl.BlockSpec((_NOUT, _D), lambda: (0, 0)),
            pl.BlockSpec((_NOUT, _D), lambda: (0, 0)),
            pl.BlockSpec((_NW, _D), lambda: (0, 0)),
            pl.BlockSpec((_NW, _D), lambda: (0, 0)),
            pl.BlockSpec((_NW, _D), lambda: (0, 0)),
        ],
        out_specs=pl.BlockSpec((_N, _D), lambda: (0, 0)),
        out_shape=jax.ShapeDtypeStruct((_N, _D), jnp.float32),
    )(num, den, side_num, side_den, side_id)


def kernel(obs_encoding, lane_encoding, same_obs_mask, W, b):
    idx = same_obs_mask[:, 0].astype(jnp.int32)
    wl = W[_D:, :]
    ex = _tc_ex(lane_encoding, wl).reshape(_M)
    num_ref = jax.new_ref(jnp.zeros((_NOUT, _D), jnp.float32))
    den_ref = jax.new_ref(jnp.zeros((_NOUT, _D), jnp.float32))
    side_num, side_den, side_id = _sc_scatter(
        lane_encoding, idx, ex, num_ref, den_ref)
    return _tc_norm(num_ref[...], den_ref[...], side_num, side_den, side_id)


# speedup vs baseline: 1.2805x; 1.2805x over previous
import functools

import jax
import jax.numpy as jnp
from jax import lax
from jax.experimental import pallas as pl
from jax.experimental.pallas import tpu as pltpu
from jax.experimental.pallas import tpu_sc as plsc

_N = 1024
_M = 16384
_D = 128
_MT = 2048
_NW = 32
_RPW = _M // _NW
_CHR = 128
_NCH = _RPW // _CHR
_NOUT = _N + _NW


def _ex_body(lane_ref, wl_ref, ex_ref):
    s = jax.lax.dot_general(lane_ref[...], wl_ref[...], (((1,), (0,)), ((), ())),
                            preferred_element_type=jnp.float32)
    ex_ref[...] = jnp.broadcast_to(jnp.exp(s), (_MT, 16))


def _tc_ex(lane, wl):
    return pl.pallas_call(
        _ex_body,
        grid=(_M // _MT,),
        in_specs=[
            pl.BlockSpec((_MT, _D), lambda i: (i, 0)),
            pl.BlockSpec((_D, 1), lambda i: (0, 0)),
        ],
        out_specs=pl.BlockSpec((_MT, 16), lambda i: (i, 0)),
        out_shape=jax.ShapeDtypeStruct((_M, 16), jnp.float32),
    )(lane, wl)


_sc_mesh = plsc.VectorSubcoreMesh(core_axis_name="c", subcore_axis_name="s")


@functools.partial(
    pl.kernel,
    out_type=(jax.ShapeDtypeStruct((_NW, _D), jnp.float32),
              jax.ShapeDtypeStruct((_NW, _D), jnp.float32),
              jax.ShapeDtypeStruct((_NW, _D), jnp.int32)),
    mesh=_sc_mesh,
    compiler_params=pltpu.CompilerParams(needs_layout_passes=False,
                                         use_tc_tiling_on_sc=False),
    scratch_types=[
        pltpu.VMEM((_CHR, _D), jnp.float32),
        pltpu.VMEM((_CHR, _D), jnp.float32),
        pltpu.VMEM((_RPW, 16), jnp.float32),
        pltpu.VMEM((768,), jnp.int32),
        pltpu.VMEM((_RPW,), jnp.int32),
        pltpu.VMEM((768,), jnp.int32),
        pltpu.VMEM((_RPW,), jnp.int32),
        pltpu.VMEM((4, _D), jnp.int32),
        pltpu.VMEM((_RPW, _D), jnp.float32),
        pltpu.VMEM((_RPW, 16), jnp.float32),
        pltpu.VMEM((1, _D), jnp.float32),
        pltpu.VMEM((1, _D), jnp.int32),
        pltpu.SemaphoreType.DMA,
        pltpu.SemaphoreType.DMA,
        pltpu.SemaphoreType.DMA,
    ],
)
def _sc_scatter(lane_hbm, idx_hbm, ex_hbm, num_ref, den_ref,
                side_num, side_den, side_id,
                lane0_b, lane1_b, ex_b, idxe_b, segs_b, starts_b, ids_b,
                ids2_b, locn_b, locd_b, tmpf_b, tmpi_b, semA, semB, semC):
    c = lax.axis_index("c")
    s = lax.axis_index("s")
    w = c * 16 + s
    base = w * _RPW
    li = lax.iota(jnp.int32, 16)
    bufs = (lane0_b, lane1_b)
    sems = (semA, semB)

    handles = [None] * _NCH
    handles[0] = pltpu.async_copy(lane_hbm.at[pl.ds(base, _CHR)], lane0_b,
                                  semA)
    handles[1] = pltpu.async_copy(lane_hbm.at[pl.ds(base + _CHR, _CHR)],
                                  lane1_b, semB)
    pltpu.sync_copy(ex_hbm.at[pl.ds(base, _RPW)], ex_b)
    pltpu.sync_copy(idx_hbm.at[pl.ds(base, _RPW)], idxe_b.at[pl.ds(16, _RPW)])
    head = idxe_b[pl.ds(16, 16)]
    idxe_b[pl.ds(0, 16)] = jnp.broadcast_to(jnp.min(head), (16,))

    dumpv = jnp.full((16,), _N + w, jnp.int32)
    for k in range(_RPW // 16):
        ids_b[pl.ds(k * 16, 16)] = dumpv

    c15 = jnp.full((16,), 15, jnp.int32)
    rpwv = jnp.full((16,), _RPW, jnp.int32)
    for k in range(768 // 16):
        starts_b[pl.ds(k * 16, 16)] = rpwv
    zi = jnp.zeros((16,), jnp.int32)
    starts_b[pl.ds(0, 16)] = jnp.where(li == 0, zi, rpwv)

    def _seg(g, sbv):
        iv = idxe_b[pl.ds(16 + g * 16, 16)]
        sv = idxe_b[pl.ds(15 + g * 16, 16)]
        bnd = iv != sv
        segs = sbv + plsc.cumsum(bnd.astype(jnp.int32))
        segs_b[pl.ds(g * 16, 16)] = segs
        plsc.store_scatter(ids_b, [segs], iv)
        plsc.store_scatter(starts_b, [segs], g * 16 + li, mask=bnd)
        return jnp.take_along_axis(segs, c15, axis=0)

    sbv = lax.fori_loop(0, _RPW // 16, _seg, jnp.zeros((16,), jnp.int32))
    nseg = jnp.max(sbv) + 1

    zv = jnp.zeros((16,), jnp.float32)

    def _zr(r, carry):
        for dd in range(_D // 16):
            locn_b[r, pl.ds(dd * 16, 16)] = zv
        locd_b[r, pl.ds(0, 16)] = zv
        return carry

    lax.fori_loop(0, nseg, _zr, 0)

    zacc = jnp.zeros((16,), jnp.float32)
    for ci in range(_NCH):
        handles[ci].wait()
        buf = bufs[ci % 2]
        lob = ci * _CHR
        s_lo = jnp.min(segs_b[pl.ds(lob, 16)])
        s_hi = jnp.max(segs_b[pl.ds(lob + _CHR - 16, 16)])

        def _seg_loop(sgi, carry, buf=buf, lob=lob):
            st = jnp.maximum(jnp.min(starts_b[pl.ds(sgi, 16)]), lob)
            en = jnp.minimum(jnp.min(starts_b[pl.ds(sgi + 1, 16)]),
                             lob + _CHR)

            def _row(r, accs):
                evr = ex_b[r, pl.ds(0, 16)]
                out = []
                for dd in range(_D // 16):
                    v = buf[r - lob, pl.ds(dd * 16, 16)]
                    out.append(accs[dd] + v * evr)
                out.append(accs[_D // 16] + evr)
                return tuple(out)

            accs = lax.fori_loop(st, en, _row, (zacc,) * (_D // 16 + 1))
            for dd in range(_D // 16):
                locn_b[sgi, pl.ds(dd * 16, 16)] = (
                    locn_b[sgi, pl.ds(dd * 16, 16)] + accs[dd])
            locd_b[sgi, pl.ds(0, 16)] = (
                locd_b[sgi, pl.ds(0, 16)] + accs[_D // 16])
            return carry

        lax.fori_loop(s_lo, s_hi + 1, _seg_loop, 0)
        if ci + 2 < _NCH:
            handles[ci + 2] = pltpu.async_copy(
                lane_hbm.at[pl.ds(base + (ci + 2) * _CHR, _CHR)],
                bufs[ci % 2], sems[ci % 2])

    zv128 = jnp.zeros((16,), jnp.float32)
    for dd in range(_D // 16):
        tmpf_b[0, pl.ds(dd * 16, 16)] = zv128
        tmpi_b[0, pl.ds(dd * 16, 16)] = jnp.zeros((16,), jnp.int32)
    hside_n = pltpu.async_copy(locn_b.at[pl.ds(0, 1)],
                               side_num.at[pl.ds(w, 1)], semC)
    tmpf_b[0, pl.ds(0, 16)] = locd_b[0, pl.ds(0, 16)]
    hside_d = pltpu.async_copy(tmpf_b, side_den.at[pl.ds(w, 1)], semC)
    idv = ids_b[pl.ds(0, 16)]
    tmpi_b[0, pl.ds(0, 16)] = idv
    hside_i = pltpu.async_copy(tmpi_b, side_id.at[pl.ds(w, 1)], semC)
    ids_b[pl.ds(0, 16)] = jnp.where(li == 0, dumpv, idv)

    for k in range(_RPW // 16):
        ids2_b[k // 8, pl.ds((k % 8) * 16, 16)] = ids_b[pl.ds(k * 16, 16)]

    def _cp0(r, carry):
        lane0_b[r, pl.ds(0, 16)] = locd_b[r, pl.ds(0, 16)]
        return carry

    lax.fori_loop(0, 128, _cp0, 0)
    hnum = pltpu.async_copy(locn_b.at[pl.ds(0, 128)],
                            num_ref.at[ids2_b.at[0]], semC)
    hden = pltpu.async_copy(lane0_b, den_ref.at[ids2_b.at[0]], semC)

    for k in range(1, 4):
        @pl.when(nseg > k * 128)
        def _(k=k):
            pltpu.sync_copy(locn_b.at[pl.ds(k * 128, 128)],
                            num_ref.at[ids2_b.at[k]])

            def _cp(r, carry):
                lane1_b[r, pl.ds(0, 16)] = locd_b[k * 128 + r, pl.ds(0, 16)]
                return carry

            lax.fori_loop(0, 128, _cp, 0)
            pltpu.sync_copy(lane1_b, den_ref.at[ids2_b.at[k]])

    hside_n.wait()
    hside_d.wait()
    hside_i.wait()
    hnum.wait()
    hden.wait()


def _norm_body(num_ref, den_ref, sn_ref, sd_ref, si_ref, out_ref):
    num = num_ref[...][:_N]
    den = den_ref[...][:_N, 0:1]
    sid = si_ref[...][:, 0:1]
    obs = lax.broadcasted_iota(jnp.int32, (_N, _NW), 0)
    sidb = jnp.broadcast_to(sid.T, (_N, _NW))
    sel = jnp.where(obs == sidb, 1.0, 0.0)
    num_tot = num + jax.lax.dot_general(
        sel, sn_ref[...], (((1,), (0,)), ((), ())),
        preferred_element_type=jnp.float32)
    den_tot = den + jax.lax.dot_general(
        sel, sd_ref[...], (((1,), (0,)), ((), ())),
        preferred_element_type=jnp.float32)[:, 0:1]
    ok = den_tot > 0.0
    denq = jnp.where(ok, den_tot, 1.0)
    out_ref[...] = jnp.where(jnp.broadcast_to(ok, (_N, _D)),
                             num_tot / jnp.broadcast_to(denq, (_N, _D)),
                             0.0)


def _tc_norm(num, den, side_num, side_den, side_id):
    return pl.pallas_call(
        _norm_body,
        in_specs=[
            pl.BlockSpec((_NOUT, _D), lambda: (0, 0)),
            pl.BlockSpec((_NOUT, _D), lambda: (0, 0)),
            pl.BlockSpec((_NW, _D), lambda: (0, 0)),
            pl.BlockSpec((_NW, _D), lambda: (0, 0)),
            pl.BlockSpec((_NW, _D), lambda: (0, 0)),
        ],
        out_specs=pl.BlockSpec((_N, _D), lambda: (0, 0)),
        out_shape=jax.ShapeDtypeStruct((_N, _D), jnp.float32),
    )(num, den, side_num, side_den, side_id)


def kernel(obs_encoding, lane_encoding, same_obs_mask, W, b):
    idx = same_obs_mask[:, 0].astype(jnp.int32)
    wl = W[_D:, :]
    ex = _tc_ex(lane_encoding, wl)
    num_ref = jax.new_ref(jnp.zeros((_NOUT, _D), jnp.float32))
    den_ref = jax.new_ref(jnp.zeros((_NOUT, _D), jnp.float32))
    side_num, side_den, side_id = _sc_scatter(
        lane_encoding, idx, ex, num_ref, den_ref)
    return _tc_norm(num_ref[...], den_ref[...], side_num, side_den, side_id)
